# baseline (speedup 1.0000x reference)
import functools

import jax
import jax.numpy as jnp
from jax import lax
from jax.experimental import pallas as pl
from jax.experimental.pallas import tpu as pltpu
from jax.experimental.pallas import tpu_sc as plsc

_NC, _NS = 2, 16
_NW = _NC * _NS
_L16 = 16


@functools.cache
def _build_pairpack(v, d):
    blk = 1000
    nblk = v // blk

    def body(lo_ref, hi_ref, out_ref):
        out_ref[:, :d] = lo_ref[...]
        out_ref[:, d:] = hi_ref[...]

    return pl.pallas_call(
        body,
        grid=(nblk,),
        in_specs=[
            pl.BlockSpec((blk, d), lambda i: (i, 0)),
            pl.BlockSpec((blk, d), lambda i: (lax.rem(i + nblk // 2, nblk), 0)),
        ],
        out_specs=pl.BlockSpec((blk, 2 * d), lambda i: (i, 0)),
        out_shape=jax.ShapeDtypeStruct((v, 2 * d), jnp.float32),
    )


@functools.cache
def _build_gather(b, l, d):
    seq_per_w = b // _NW
    cpad = l + (-l % _L16)
    mesh = plsc.VectorSubcoreMesh(core_axis_name="c", subcore_axis_name="s")

    @functools.partial(
        pl.kernel,
        out_type=jax.ShapeDtypeStruct((b, l, d), jnp.float32),
        mesh=mesh,
        scratch_types=[
            pltpu.VMEM((cpad,), jnp.int32),
            pltpu.VMEM((cpad,), jnp.int32),
            pltpu.VMEM((cpad, 2 * d), jnp.float32),
            pltpu.VMEM((cpad, 2 * d), jnp.float32),
            pltpu.VMEM((l, d), jnp.float32),
            pltpu.VMEM((l, d), jnp.float32),
            pltpu.SemaphoreType.DMA,
            pltpu.SemaphoreType.DMA,
            pltpu.SemaphoreType.DMA,
            pltpu.SemaphoreType.DMA,
            pltpu.SemaphoreType.DMA,
            pltpu.SemaphoreType.DMA,
        ],
        compiler_params=pltpu.CompilerParams(needs_layout_passes=False),
    )
    def gather(idx_hbm, table3_hbm, out_hbm,
               idx0, idx1, rows0, rows1, sel0, sel1,
               isem0, isem1, gsem0, gsem1, wsem0, wsem1):
        wid = lax.axis_index("s") * _NC + lax.axis_index("c")
        seq0 = wid * seq_per_w
        idx_v = (idx0, idx1)
        rows_v = (rows0, rows1)
        sel_v = (sel0, sel1)
        isem = (isem0, isem1)
        gsem = (gsem0, gsem1)
        wsem = (wsem0, wsem1)

        def idx_start(i, u):
            pltpu.async_copy(
                idx_hbm.at[pl.ds((seq0 + i) * l, l)],
                idx_v[u].at[pl.ds(0, l)], isem[u])

        def idx_wait(u):
            pltpu.make_async_copy(
                idx_hbm.at[pl.ds(0, l)], idx_v[u].at[pl.ds(0, l)],
                isem[u]).wait()

        def gather_start(u):
            pltpu.async_copy(table3_hbm.at[idx_v[u]], rows_v[u], gsem[u])

        def gather_wait(u):
            pltpu.make_async_copy(
                table3_hbm.at[idx_v[u]], rows_v[u], gsem[u]).wait()

        def extract(u):
            @pl.loop(0, l)
            def _(i):
                for g in range(d // _L16):
                    sel_v[u][i, pl.ds(g * _L16, _L16)] = (
                        rows_v[u][i, pl.ds(g * _L16, _L16)])

        def write_start(i, u):
            pltpu.async_copy(sel_v[u], out_hbm.at[seq0 + i], wsem[u])

        def write_wait(u):
            pltpu.make_async_copy(sel_v[u], out_hbm.at[0], wsem[u]).wait()

        zeros = jnp.zeros((_L16,), jnp.int32)
        for u in (0, 1):
            @pl.loop(l // _L16, cpad // _L16)
            def _(j):
                idx_v[u][pl.ds(j * _L16, _L16)] = zeros

        idx_start(0, 0)
        idx_start(1, 1)
        idx_wait(0)
        gather_start(0)

        @pl.loop(0, seq_per_w // 2)
        def _outer(j):
            for u in (0, 1):
                i = j * 2 + u
                nu = 1 - u
                gather_wait(u)

                @pl.when(i + 2 < seq_per_w)
                def _():
                    idx_start(i + 2, u)

                @pl.when(i + 1 < seq_per_w)
                def _():
                    idx_wait(nu)
                    gather_start(nu)

                @pl.when(i >= 2)
                def _():
                    write_wait(u)
                extract(u)
                write_start(i, u)

        write_wait(0)
        write_wait(1)

    return gather


def kernel(sequences_tokens, embedding_table):
    b, l = sequences_tokens.shape
    v, d = embedding_table.shape
    idx = sequences_tokens.reshape(b * l)
    table3 = _build_pairpack(v, d)(embedding_table, embedding_table)
    return _build_gather(b, l, d)(idx, table3)


# speedup vs baseline: 2.4727x; 2.4727x over previous
import functools

import jax
import jax.numpy as jnp
from jax import lax
from jax.experimental import pallas as pl
from jax.experimental.pallas import tpu as pltpu
from jax.experimental.pallas import tpu_sc as plsc

_NC, _NS = 2, 16
_NW = _NC * _NS
_CHUNK = 800


_PBLK = 1000


@functools.cache
def _build_pack(v, d):
    blk = _PBLK

    def body(in_ref, out_ref):
        x = in_ref[...]
        out_ref[:, :d] = x[:blk]
        out_ref[:, d:] = x[blk:]

    return pl.pallas_call(
        body,
        grid=(v // (2 * blk),),
        in_specs=[pl.BlockSpec((2 * blk, d), lambda i: (i, 0))],
        out_specs=pl.BlockSpec((blk, 2 * d), lambda i: (i, 0)),
        out_shape=jax.ShapeDtypeStruct((v // 2, 2 * d), jnp.float32),
    )


@functools.cache
def _build_gather(b, l, d):
    n = b * l
    n_per_w = n // _NW
    n_chunks = n_per_w // _CHUNK
    assert n_chunks % 2 == 0 and n_chunks >= 4
    mesh = plsc.VectorSubcoreMesh(core_axis_name="c", subcore_axis_name="s")

    @functools.partial(
        pl.kernel,
        out_type=jax.ShapeDtypeStruct((b, l, d), jnp.float32),
        mesh=mesh,
        scratch_types=[
            pltpu.VMEM((_CHUNK,), jnp.int32),
            pltpu.VMEM((_CHUNK,), jnp.int32),
            pltpu.VMEM((_CHUNK, d), jnp.float32),
            pltpu.VMEM((_CHUNK, d), jnp.float32),
            pltpu.SemaphoreType.DMA,
            pltpu.SemaphoreType.DMA,
            pltpu.SemaphoreType.DMA,
            pltpu.SemaphoreType.DMA,
            pltpu.SemaphoreType.DMA,
            pltpu.SemaphoreType.DMA,
        ],
        compiler_params=pltpu.CompilerParams(use_tc_tiling_on_sc=False),
    )
    def gather(idx_hbm, table_hbm, out3_hbm,
               idx0, idx1, rows0, rows1,
               isem0, isem1, gsem0, gsem1, wsem0, wsem1):
        seq_per_chunk = _CHUNK // l
        wid = lax.axis_index("s") * _NC + lax.axis_index("c")
        base = wid * n_per_w
        seq_base = wid * (n_per_w // l)
        idx_v = (idx0, idx1)
        rows_v = (rows0, rows1)
        isem = (isem0, isem1)
        gsem = (gsem0, gsem1)
        wsem = (wsem0, wsem1)

        def idx_start(i, u):
            pltpu.async_copy(
                idx_hbm.at[pl.ds(base + i * _CHUNK, _CHUNK)], idx_v[u], isem[u])

        def idx_wait(u):
            pltpu.make_async_copy(
                idx_hbm.at[pl.ds(0, _CHUNK)], idx_v[u], isem[u]).wait()

        def gather_start(u):
            pltpu.async_copy(table_hbm.at[idx_v[u]], rows_v[u], gsem[u])

        def gather_wait(u):
            pltpu.make_async_copy(
                table_hbm.at[idx_v[u]], rows_v[u], gsem[u]).wait()

        def write_start(i, u):
            for k in range(seq_per_chunk):
                pltpu.async_copy(
                    rows_v[u].at[pl.ds(k * l, l)],
                    out3_hbm.at[seq_base + i * seq_per_chunk + k], wsem[u])

        def write_wait(u):
            for k in range(seq_per_chunk):
                pltpu.make_async_copy(
                    rows_v[u].at[pl.ds(k * l, l)], out3_hbm.at[0],
                    wsem[u]).wait()

        idx_start(0, 0)
        idx_start(1, 1)
        idx_wait(0)
        gather_start(0)

        @pl.loop(0, n_chunks // 2)
        def _outer(j):
            for u in (0, 1):
                i = j * 2 + u
                nu = 1 - u
                gather_wait(u)

                @pl.when(i + 2 < n_chunks)
                def _():
                    idx_start(i + 2, u)

                @pl.when(i + 1 < n_chunks)
                def _():
                    idx_wait(nu)

                    @pl.when(i >= 1)
                    def _():
                        write_wait(nu)
                    gather_start(nu)

                write_start(i, u)

        write_wait(0)
        write_wait(1)

    return gather


def kernel(sequences_tokens, embedding_table):
    b, l = sequences_tokens.shape
    v, d = embedding_table.shape
    idx = sequences_tokens.reshape(b * l)
    t2 = _build_pack(v, d)(embedding_table)
    t_view = t2.reshape(v, d)
    blk2 = 2 * _PBLK
    i_blk = idx // blk2
    rem = idx - i_blk * blk2
    midx = 2 * (i_blk * _PBLK + jnp.where(rem < _PBLK, rem, rem - _PBLK)) + (
        rem >= _PBLK)
    return _build_gather(b, l, d)(midx, t_view)
